# Initial kernel scaffold; baseline (speedup 1.0000x reference)
import jax
import jax.numpy as jnp
from jax.experimental import pallas as pl


def kernel(x, edge_index, W1, b1, W2, b2):
    raise NotImplementedError("write your pallas kernel here")


# speedup vs baseline: 14.9218x; 14.9218x over previous
import functools

import jax
import jax.numpy as jnp
from jax import lax
from jax.experimental import pallas as pl
from jax.experimental.pallas import tpu as pltpu
from jax.experimental.pallas import tpu_sc as plsc

N = 10000
NP = 10240
E = 160000
EP = 163840
F = 256
FH = 128
C = 40
CP = 128
BN = 1000
NBLK = N // BN

_mesh = plsc.VectorSubcoreMesh(core_axis_name="c", subcore_axis_name="s")


def _fill(ref, rows, cols, value):
    @pl.loop(0, rows)
    def _(i):
        @pl.loop(0, cols, step=16)
        def _(j):
            ref[i, pl.ds(j, 16)] = jnp.full((16,), value, jnp.float32)


def _fill1d(ref, n, value):
    @pl.loop(0, n, step=16)
    def _(j):
        ref[pl.ds(j, 16)] = jnp.full((16,), value, jnp.float32)


@functools.partial(
    pl.kernel, mesh=_mesh,
    out_type=jax.ShapeDtypeStruct((2, NP), jnp.float32),
    scratch_types=[
        pltpu.VMEM((EP // 128 // 32, 128), jnp.int32),
        pltpu.VMEM((128,), jnp.float32),
        pltpu.VMEM((640,), jnp.float32),
        pltpu.VMEM_SHARED((NP,), jnp.float32),
    ],
)
def _sc_degree(dst_hbm, out_hbm, idx_v, ones_v, zeros_v, deg_sh):
    c = lax.axis_index("c")
    s = lax.axis_index("s")
    w = c * 16 + s
    rw = EP // 128 // 32
    _fill1d(ones_v, 128, 1.0)
    _fill1d(zeros_v, 640, 0.0)
    pltpu.sync_copy(zeros_v, deg_sh.at[pl.ds(s * 640, 640)])
    pltpu.sync_copy(dst_hbm.at[pl.ds(w * rw, rw)], idx_v)
    plsc.subcore_barrier()

    @pl.loop(0, rw)
    def _(k):
        pltpu.sync_copy(ones_v, deg_sh.at[idx_v.at[k]], add=True)

    plsc.subcore_barrier()

    @pl.when(c == 0)
    def _():
        pltpu.sync_copy(deg_sh.at[pl.ds(s * 640, 640)],
                        out_hbm.at[0, pl.ds(s * 640, 640)])

    @pl.when(c == 1)
    def _():
        pltpu.sync_copy(deg_sh.at[pl.ds(s * 640, 640)],
                        out_hbm.at[1, pl.ds(s * 640, 640)])


@functools.partial(
    pl.kernel, mesh=_mesh,
    out_type=jax.ShapeDtypeStruct((2, NP, FH), jnp.float32),
    scratch_types=[
        pltpu.VMEM((EP // 128 // 16, 128), jnp.int32),
        pltpu.VMEM((EP // 128 // 16, 128), jnp.int32),
        pltpu.VMEM((128, FH), jnp.float32),
        pltpu.VMEM((16, FH), jnp.float32),
        pltpu.VMEM_SHARED((NP, FH), jnp.float32),
    ],
)
def _sc_edge256(sa_hbm, sb_hbm, src_hbm, dst_hbm, out_hbm,
                src_v, dst_v, rows_v, zero_v, acc_sh):
    c = lax.axis_index("c")
    s = lax.axis_index("s")
    rw = EP // 128 // 16
    _fill(zero_v, 16, FH, 0.0)

    @pl.loop(0, 40)
    def _(t):
        pltpu.sync_copy(zero_v, acc_sh.at[pl.ds(s * 640 + t * 16, 16)])

    pltpu.sync_copy(src_hbm.at[pl.ds(s * rw, rw)], src_v)
    pltpu.sync_copy(dst_hbm.at[pl.ds(s * rw, rw)], dst_v)
    plsc.subcore_barrier()

    def edge_pass(table_hbm):
        @pl.loop(0, rw)
        def _(k):
            pltpu.sync_copy(table_hbm.at[src_v.at[k]], rows_v)
            pltpu.sync_copy(rows_v, acc_sh.at[dst_v.at[k]], add=True)

    @pl.when(c == 0)
    def _():
        edge_pass(sa_hbm)

    @pl.when(c == 1)
    def _():
        edge_pass(sb_hbm)

    plsc.subcore_barrier()

    @pl.when(c == 0)
    def _():
        pltpu.sync_copy(acc_sh.at[pl.ds(s * 640, 640)],
                        out_hbm.at[0, pl.ds(s * 640, 640)])

    @pl.when(c == 1)
    def _():
        pltpu.sync_copy(acc_sh.at[pl.ds(s * 640, 640)],
                        out_hbm.at[1, pl.ds(s * 640, 640)])


@functools.partial(
    pl.kernel, mesh=_mesh,
    out_type=jax.ShapeDtypeStruct((2, NP, CP), jnp.float32),
    scratch_types=[
        pltpu.VMEM((EP // 128 // 32, 128), jnp.int32),
        pltpu.VMEM((EP // 128 // 32, 128), jnp.int32),
        pltpu.VMEM((128, CP), jnp.float32),
        pltpu.VMEM((16, CP), jnp.float32),
        pltpu.VMEM_SHARED((NP, CP), jnp.float32),
    ],
)
def _sc_edge64(s2_hbm, src_hbm, dst_hbm, out_hbm,
               src_v, dst_v, rows_v, zero_v, acc_sh):
    c = lax.axis_index("c")
    s = lax.axis_index("s")
    w = c * 16 + s
    rw = EP // 128 // 32
    _fill(zero_v, 16, CP, 0.0)

    @pl.loop(0, 40)
    def _(t):
        pltpu.sync_copy(zero_v, acc_sh.at[pl.ds(s * 640 + t * 16, 16)])

    pltpu.sync_copy(src_hbm.at[pl.ds(w * rw, rw)], src_v)
    pltpu.sync_copy(dst_hbm.at[pl.ds(w * rw, rw)], dst_v)
    plsc.subcore_barrier()

    @pl.loop(0, rw)
    def _(k):
        pltpu.sync_copy(s2_hbm.at[src_v.at[k]], rows_v)
        pltpu.sync_copy(rows_v, acc_sh.at[dst_v.at[k]], add=True)

    plsc.subcore_barrier()

    @pl.when(c == 0)
    def _():
        pltpu.sync_copy(acc_sh.at[pl.ds(s * 640, 640)],
                        out_hbm.at[0, pl.ds(s * 640, 640)])

    @pl.when(c == 1)
    def _():
        pltpu.sync_copy(acc_sh.at[pl.ds(s * 640, 640)],
                        out_hbm.at[1, pl.ds(s * 640, 640)])


def _tc_l1_body(x_ref, w_ref, da_ref, db_ref, sa_ref, sb_ref, dinv_ref):
    dinv = lax.rsqrt(jnp.maximum(da_ref[...] + db_ref[...] + 1.0, 1e-12))
    xw = jnp.dot(x_ref[...], w_ref[...], preferred_element_type=jnp.float32)
    scaled = xw * dinv
    sa_ref[...] = scaled[:, :FH]
    sb_ref[...] = scaled[:, FH:]
    dinv_ref[...] = dinv


def _tc_l1(x, w1, da, db):
    return pl.pallas_call(
        _tc_l1_body,
        grid=(NBLK,),
        in_specs=[
            pl.BlockSpec((BN, F), lambda i: (i, 0)),
            pl.BlockSpec((F, F), lambda i: (0, 0)),
            pl.BlockSpec((BN, 1), lambda i: (i, 0)),
            pl.BlockSpec((BN, 1), lambda i: (i, 0)),
        ],
        out_specs=[
            pl.BlockSpec((BN, FH), lambda i: (i, 0)),
            pl.BlockSpec((BN, FH), lambda i: (i, 0)),
            pl.BlockSpec((BN, 1), lambda i: (i, 0)),
        ],
        out_shape=[
            jax.ShapeDtypeStruct((N, FH), jnp.float32),
            jax.ShapeDtypeStruct((N, FH), jnp.float32),
            jax.ShapeDtypeStruct((N, 1), jnp.float32),
        ],
    )(x, w1, da, db)


def _tc_l2_body(aa_ref, ab_ref, sa_ref, sb_ref, dinv_ref, b1_ref, w2_ref,
                out_ref):
    dinv = dinv_ref[...]
    b1 = b1_ref[...]
    ha = jnp.maximum(dinv * (aa_ref[0] + sa_ref[...]) + b1[:, :FH], 0.0)
    hb = jnp.maximum(dinv * (ab_ref[0] + sb_ref[...]) + b1[:, FH:], 0.0)
    h = jnp.concatenate([ha, hb], axis=1)
    hw = jnp.dot(h, w2_ref[...], preferred_element_type=jnp.float32)
    out_ref[...] = hw * dinv


def _tc_l2(agg, sa, sb, dinv, b1, w2p):
    return pl.pallas_call(
        _tc_l2_body,
        grid=(NBLK,),
        in_specs=[
            pl.BlockSpec((1, BN, FH), lambda i: (0, i, 0)),
            pl.BlockSpec((1, BN, FH), lambda i: (1, i, 0)),
            pl.BlockSpec((BN, FH), lambda i: (i, 0)),
            pl.BlockSpec((BN, FH), lambda i: (i, 0)),
            pl.BlockSpec((BN, 1), lambda i: (i, 0)),
            pl.BlockSpec((1, F), lambda i: (0, 0)),
            pl.BlockSpec((F, CP), lambda i: (0, 0)),
        ],
        out_specs=pl.BlockSpec((BN, CP), lambda i: (i, 0)),
        out_shape=jax.ShapeDtypeStruct((N, CP), jnp.float32),
    )(agg, agg, sa, sb, dinv, b1, w2p)


def _tc_out_body(aa_ref, ab_ref, s2_ref, dinv_ref, b2_ref, out_ref):
    t = dinv_ref[...] * (aa_ref[0] + ab_ref[0] + s2_ref[...]) + b2_ref[...]
    t = t[:, :C]
    m = jnp.max(t, axis=1, keepdims=True)
    lse = jnp.log(jnp.sum(jnp.exp(t - m), axis=1, keepdims=True)) + m
    out_ref[...] = t - lse


def _tc_out(agg2, s2, dinv, b2p):
    return pl.pallas_call(
        _tc_out_body,
        grid=(NBLK,),
        in_specs=[
            pl.BlockSpec((1, BN, CP), lambda i: (0, i, 0)),
            pl.BlockSpec((1, BN, CP), lambda i: (1, i, 0)),
            pl.BlockSpec((BN, CP), lambda i: (i, 0)),
            pl.BlockSpec((BN, 1), lambda i: (i, 0)),
            pl.BlockSpec((1, CP), lambda i: (0, 0)),
        ],
        out_specs=pl.BlockSpec((BN, C), lambda i: (i, 0)),
        out_shape=jax.ShapeDtypeStruct((N, C), jnp.float32),
    )(agg2, agg2, s2, dinv, b2p)


def kernel(x, edge_index, W1, b1, W2, b2):
    pad = EP - E
    pad_src = jnp.arange(pad, dtype=jnp.int32) % N
    pad_dst = N + jnp.arange(pad, dtype=jnp.int32) % (NP - N)
    src2d = jnp.concatenate([edge_index[0], pad_src]).reshape(EP // 128, 128)
    dst2d = jnp.concatenate([edge_index[1], pad_dst]).reshape(EP // 128, 128)

    deg2 = _sc_degree(dst2d)
    da = deg2[0, :N, None]
    db = deg2[1, :N, None]
    sa, sb, dinv = _tc_l1(x, W1, da, db)
    agg = _sc_edge256(sa, sb, src2d, dst2d)
    w2p = jnp.pad(W2, ((0, 0), (0, CP - C)))
    s2 = _tc_l2(agg, sa, sb, dinv, b1[None, :], w2p)
    agg2 = _sc_edge64(s2, src2d, dst2d)
    b2p = jnp.pad(b2, (0, CP - C))[None, :]
    return _tc_out(agg2, s2, dinv, b2p)
